# Initial kernel scaffold; baseline (speedup 1.0000x reference)
import jax
import jax.numpy as jnp
from jax.experimental import pallas as pl


def kernel(x, table):
    raise NotImplementedError("write your pallas kernel here")


# speedup vs baseline: 1.3014x; 1.3014x over previous
import functools

import jax
import jax.numpy as jnp
from jax import lax
from jax.experimental import pallas as pl
from jax.experimental.pallas import tpu as pltpu
from jax.experimental.pallas import tpu_sc as plsc

ROWS, COLS = 16384, 200
VOCAB, D = 25, 256
B = ROWS * COLS
NC, NS = 2, 16
NW = NC * NS
B_PER_W = B // NW
CHUNK = 128
NCHUNK = B_PER_W // CHUNK


@functools.partial(
    pl.kernel,
    out_type=jax.ShapeDtypeStruct((B, D), jnp.float32),
    mesh=plsc.VectorSubcoreMesh(
        core_axis_name="c", subcore_axis_name="s", num_cores=NC, num_subcores=NS
    ),
    scratch_types=[
        pltpu.VMEM((CHUNK,), jnp.int32),
        pltpu.VMEM((CHUNK, D), jnp.float32),
        pltpu.SemaphoreType.DMA,
    ],
)
def _embed_gather(idx_hbm, table_hbm, out_hbm, idx_v, rows_v, sem):
    wid = lax.axis_index("s") * NC + lax.axis_index("c")
    base = wid * B_PER_W

    def body(i, carry):
        off = base + i * CHUNK
        pltpu.sync_copy(idx_hbm.at[pl.ds(off, CHUNK)], idx_v)
        pltpu.async_copy(table_hbm.at[idx_v], rows_v, sem).wait()
        pltpu.sync_copy(rows_v, out_hbm.at[pl.ds(off, CHUNK), :])
        return carry

    lax.fori_loop(0, NCHUNK, body, 0)


def kernel(x, table):
    idx = x.reshape(B)
    out = _embed_gather(idx, table)
    return out.reshape(ROWS, COLS, D)


# speedup vs baseline: 1.3103x; 1.0068x over previous
import functools

import jax
import jax.numpy as jnp
from jax import lax
from jax.experimental import pallas as pl
from jax.experimental.pallas import tpu as pltpu
from jax.experimental.pallas import tpu_sc as plsc

ROWS, COLS = 16384, 200
VOCAB, D = 25, 256
B = ROWS * COLS
NC, NS = 2, 16
NW = NC * NS
B_PER_W = B // NW
CHUNK = 128
NCHUNK = B_PER_W // CHUNK
NBUF = 2
OUTER = NCHUNK // NBUF


@functools.partial(
    pl.kernel,
    out_type=jax.ShapeDtypeStruct((B, D), jnp.float32),
    mesh=plsc.VectorSubcoreMesh(
        core_axis_name="c", subcore_axis_name="s", num_cores=NC, num_subcores=NS
    ),
    scratch_types=[
        pltpu.VMEM((NBUF, CHUNK), jnp.int32),
        pltpu.VMEM((NBUF, CHUNK, D), jnp.float32),
    ]
    + [pltpu.SemaphoreType.DMA] * (3 * NBUF),
)
def _embed_gather(idx_hbm, table_hbm, out_hbm, idx_v, rows_v, *sems):
    sem_i = sems[0:NBUF]
    sem_g = sems[NBUF : 2 * NBUF]
    sem_s = sems[2 * NBUF : 3 * NBUF]
    wid = lax.axis_index("s") * NC + lax.axis_index("c")
    base = wid * B_PER_W

    def idx_src(chunk):
        return idx_hbm.at[pl.ds(base + chunk * CHUNK, CHUNK)]

    for b in range(NBUF):
        pltpu.async_copy(idx_src(b), idx_v.at[b], sem_i[b])

    def outer(t, carry):
        for b in range(NBUF):
            i = t * NBUF + b
            off = base + i * CHUNK

            @pl.when(t > 0)
            def _wait_prev_scatter():
                pltpu.make_async_copy(
                    rows_v.at[b], out_hbm.at[pl.ds(base, CHUNK), :], sem_s[b]
                ).wait()

            pltpu.make_async_copy(idx_src(0), idx_v.at[b], sem_i[b]).wait()

            gather = pltpu.async_copy(
                table_hbm.at[idx_v.at[b]], rows_v.at[b], sem_g[b]
            )
            gather.wait()
            pltpu.async_copy(
                rows_v.at[b], out_hbm.at[pl.ds(off, CHUNK), :], sem_s[b]
            )
            nxt = jnp.minimum(i + NBUF, NCHUNK - 1)
            pltpu.async_copy(idx_src(nxt), idx_v.at[b], sem_i[b])
        return carry

    lax.fori_loop(0, OUTER, outer, 0)

    for b in range(NBUF):
        pltpu.make_async_copy(idx_src(0), idx_v.at[b], sem_i[b]).wait()
        pltpu.make_async_copy(
            rows_v.at[b], out_hbm.at[pl.ds(base, CHUNK), :], sem_s[b]
        ).wait()


def kernel(x, table):
    idx = x.reshape(B)
    out = _embed_gather(idx, table)
    return out.reshape(ROWS, COLS, D)


# speedup vs baseline: 2.7563x; 2.1036x over previous
import functools

import jax
import jax.numpy as jnp
from jax import lax
from jax.experimental import pallas as pl
from jax.experimental.pallas import tpu as pltpu
from jax.experimental.pallas import tpu_sc as plsc

ROWS, COLS = 16384, 200
VOCAB, D = 25, 256
LANES = 16
B = ROWS * COLS
NC, NS = 2, 16
NW = NC * NS
B_PER_W = B // NW
CHUNK = 128
NCHUNK = B_PER_W // CHUNK
NBUF = 2
OUTER = NCHUNK // NBUF


@functools.partial(
    pl.kernel,
    out_type=jax.ShapeDtypeStruct((B, D), jnp.float32),
    mesh=plsc.VectorSubcoreMesh(
        core_axis_name="c", subcore_axis_name="s", num_cores=NC, num_subcores=NS
    ),
    scratch_types=[
        pltpu.VMEM((NBUF, CHUNK), jnp.int32),
        pltpu.VMEM((NBUF, CHUNK, D), jnp.float32),
        pltpu.VMEM((VOCAB, D), jnp.float32),
    ]
    + [pltpu.SemaphoreType.DMA] * (2 * NBUF),
)
def _embed_expand(idx_hbm, table_hbm, out_hbm, idx_v, rows_v, table_v, *sems):
    sem_i = sems[0:NBUF]
    sem_s = sems[NBUF : 2 * NBUF]
    wid = lax.axis_index("s") * NC + lax.axis_index("c")
    base = wid * B_PER_W

    def idx_src(chunk):
        return idx_hbm.at[pl.ds(base + chunk * CHUNK, CHUNK)]

    pltpu.sync_copy(table_hbm, table_v)

    for b in range(NBUF):
        pltpu.async_copy(idx_src(b), idx_v.at[b], sem_i[b])

    def outer(t, carry):
        for b in range(NBUF):
            i = t * NBUF + b
            off = base + i * CHUNK

            @pl.when(t > 0)
            def _wait_prev_scatter():
                pltpu.make_async_copy(
                    rows_v.at[b], out_hbm.at[pl.ds(base, CHUNK), :], sem_s[b]
                ).wait()

            pltpu.make_async_copy(idx_src(0), idx_v.at[b], sem_i[b]).wait()

            def group(g, c2):
                r0 = g * LANES
                ivec = idx_v[b, pl.ds(r0, LANES)]
                for l in range(LANES):
                    s = ivec[l]
                    for c in range(D // LANES):
                        rows_v[b, r0 + l, pl.ds(c * LANES, LANES)] = table_v[
                            s, pl.ds(c * LANES, LANES)
                        ]
                return c2

            lax.fori_loop(0, CHUNK // LANES, group, 0)

            pltpu.async_copy(
                rows_v.at[b], out_hbm.at[pl.ds(off, CHUNK), :], sem_s[b]
            )
            nxt = jnp.minimum(i + NBUF, NCHUNK - 1)
            pltpu.async_copy(idx_src(nxt), idx_v.at[b], sem_i[b])
        return carry

    lax.fori_loop(0, OUTER, outer, 0)

    for b in range(NBUF):
        pltpu.make_async_copy(idx_src(0), idx_v.at[b], sem_i[b]).wait()
        pltpu.make_async_copy(
            rows_v.at[b], out_hbm.at[pl.ds(base, CHUNK), :], sem_s[b]
        ).wait()


def kernel(x, table):
    idx = x.reshape(B)
    out = _embed_expand(idx, table)
    return out.reshape(ROWS, COLS, D)
